# baseline (speedup 1.0000x reference)
import functools

import jax
import jax.numpy as jnp
from jax import lax
from jax.experimental import pallas as pl
from jax.experimental.pallas import tpu as pltpu

INPUT_DIM = 128
EXPERT_NUM = 1024
BATCH = 16384
AUG = INPUT_DIM + 1
TB = 512
NW = 32
CHUNKS = 4
CB = BATCH // CHUNKS
BPW = CB // NW
NCHUNK = BPW // 128


def _tc_body(x_ref, w1_ref, b1_ref, w2_ref, b2_ref, w3_ref, b3_ref,
             cbt_ref, what_ref, idx_ref, yt_ref, wl_ref):
    x = x_ref[...]
    h = jnp.maximum(jnp.dot(x, w1_ref[...],
                            preferred_element_type=jnp.float32)
                    + b1_ref[...], 0.0)
    h = jnp.maximum(jnp.dot(h, w2_ref[...],
                            preferred_element_type=jnp.float32)
                    + b2_ref[...], 0.0)
    w_hat = jnp.dot(h, w3_ref[...],
                    preferred_element_type=jnp.float32) + b3_ref[...]
    what_ref[...] = w_hat

    aug = jnp.concatenate(
        [x, jnp.ones((x.shape[0], 1), dtype=jnp.float32)], axis=1)
    scores = jnp.dot(aug, cbt_ref[...], preferred_element_type=jnp.float32)
    p = aug * w_hat
    p = jnp.concatenate([p, jnp.zeros((x.shape[0], 7), jnp.float32)], axis=1)
    g = p[:, 0:8]
    for j in range(1, 17):
        g = g + p[:, 8 * j:8 * j + 8]
    g = g[:, 0:4] + g[:, 4:8]
    g = g[:, 0:2] + g[:, 2:4]
    y_hat = g[:, 0:1] + g[:, 1:2]
    d = (y_hat - scores) ** 2
    md = jnp.min(d, axis=1, keepdims=True)
    iota = lax.broadcasted_iota(jnp.int32, d.shape, 1)
    idx = jnp.min(jnp.where(d == md, iota, jnp.int32(2 ** 30)),
                  axis=1, keepdims=True)
    idx_ref[...] = idx
    sel = iota == idx
    yt_ref[...] = jnp.sum(jnp.where(sel, scores, 0.0), axis=1, keepdims=True)
    wl_ref[...] = jnp.sum(
        jnp.where(sel, cbt_ref[INPUT_DIM:INPUT_DIM + 1, :], 0.0),
        axis=1, keepdims=True)


def _tc_call(x, W1, b1, W2, b2, W3, b3, cbt):
    grid = (CB // TB,)
    full = lambda shape: pl.BlockSpec(shape, lambda i: (0, 0))
    return pl.pallas_call(
        _tc_body,
        grid=grid,
        in_specs=[
            pl.BlockSpec((TB, INPUT_DIM), lambda i: (i, 0)),
            full((INPUT_DIM, 256)),
            full((1, 256)),
            full((256, 256)),
            full((1, 256)),
            full((256, AUG)),
            full((1, AUG)),
            full((AUG, EXPERT_NUM)),
        ],
        out_specs=[
            pl.BlockSpec((TB, AUG), lambda i: (i, 0)),
            pl.BlockSpec((TB, 1), lambda i: (i, 0)),
            pl.BlockSpec((TB, 1), lambda i: (i, 0)),
            pl.BlockSpec((TB, 1), lambda i: (i, 0)),
        ],
        out_shape=[
            jax.ShapeDtypeStruct((CB, AUG), jnp.float32),
            jax.ShapeDtypeStruct((CB, 1), jnp.int32),
            jax.ShapeDtypeStruct((CB, 1), jnp.float32),
            jax.ShapeDtypeStruct((CB, 1), jnp.float32),
        ],
    )(x, W1, b1, W2, b2, W3, b3, cbt)


def _make_sc_gather():
    from jax.experimental.pallas import tpu_sc as plsc

    info = plsc.get_sparse_core_info()
    nc = info.num_cores
    mesh = plsc.VectorSubcoreMesh(core_axis_name="c", subcore_axis_name="s")

    @functools.partial(
        pl.kernel, mesh=mesh,
        out_type=jax.ShapeDtypeStruct((CB, INPUT_DIM), jnp.float32),
        scratch_types=[
            pltpu.VMEM((NCHUNK, 128), jnp.int32),
            pltpu.VMEM((BPW, INPUT_DIM), jnp.float32),
            pltpu.SemaphoreType.DMA,
        ],
        compiler_params=pltpu.CompilerParams(use_tc_tiling_on_sc=True),
    )
    def gather_k(table_hbm, idx_hbm, out_hbm, idx_v, rows_v, sem):
        wid = lax.axis_index("s") * nc + lax.axis_index("c")
        pltpu.sync_copy(idx_hbm.at[pl.ds(wid * NCHUNK, NCHUNK)], idx_v)
        copies = [
            pltpu.async_copy(table_hbm.at[idx_v.at[j]],
                             rows_v.at[pl.ds(j * 128, 128)], sem)
            for j in range(NCHUNK)
        ]
        for c in copies:
            c.wait()
        pltpu.sync_copy(rows_v, out_hbm.at[pl.ds(wid * BPW, BPW)])

    return gather_k


def kernel(input_tensor, W1, b1, W2, b2, W3, b3, codebook):
    cbt = codebook.T
    table = codebook[:, :INPUT_DIM]
    gather = _make_sc_gather()
    b1r, b2r, b3r = b1.reshape(1, 256), b2.reshape(1, 256), b3.reshape(1, AUG)

    whs, wts, idxs, yts = [], [], [], []
    for c in range(CHUNKS):
        xc = lax.slice(input_tensor, (c * CB, 0), ((c + 1) * CB, INPUT_DIM))
        w_hat_c, idx_c, yt_c, wl_c = _tc_call(xc, W1, b1r, W2, b2r, W3,
                                              b3r, cbt)
        g_c = gather(table, idx_c.reshape(CB // 128, 128))
        whs.append(w_hat_c)
        wts.append(jnp.concatenate([g_c, wl_c], axis=1))
        idxs.append(idx_c)
        yts.append(yt_c)

    return (jnp.concatenate(whs, axis=0), jnp.concatenate(wts, axis=0),
            jnp.concatenate(idxs, axis=0), jnp.concatenate(yts, axis=0))


# speedup vs baseline: 1.6191x; 1.6191x over previous
import functools

import jax
import jax.numpy as jnp
from jax import lax
from jax.experimental import pallas as pl
from jax.experimental.pallas import tpu as pltpu

INPUT_DIM = 128
EXPERT_NUM = 1024
BATCH = 16384
AUG = INPUT_DIM + 1
TB = 512
NW = 32
CHUNKS = 1
CB = BATCH // CHUNKS
BPW = CB // NW
NCHUNK = BPW // 128


def _tc_body(x_ref, w1_ref, b1_ref, w2_ref, b2_ref, w3_ref, b3_ref,
             cbt_ref, what_ref, idx_ref, idx128_ref, yt_ref, wl_ref):
    x = x_ref[...]
    h = jnp.maximum(jnp.dot(x, w1_ref[...],
                            preferred_element_type=jnp.float32)
                    + b1_ref[...], 0.0)
    h = jnp.maximum(jnp.dot(h, w2_ref[...],
                            preferred_element_type=jnp.float32)
                    + b2_ref[...], 0.0)
    w_hat = jnp.dot(h, w3_ref[...],
                    preferred_element_type=jnp.float32) + b3_ref[...]
    what_ref[...] = w_hat

    aug = jnp.concatenate(
        [x, jnp.ones((x.shape[0], 1), dtype=jnp.float32)], axis=1)
    scores = jnp.dot(aug, cbt_ref[...], preferred_element_type=jnp.float32)
    p = aug * w_hat
    p = jnp.concatenate([p, jnp.zeros((x.shape[0], 7), jnp.float32)], axis=1)
    pt = p.T
    g = pt[0:8, :]
    for j in range(1, 17):
        g = g + pt[8 * j:8 * j + 8, :]
    g = g[0:4, :] + g[4:8, :]
    g = g[0:2, :] + g[2:4, :]
    y_hat = (g[0:1, :] + g[1:2, :]).T
    d = (y_hat - scores) ** 2
    md = jnp.min(d, axis=1, keepdims=True)
    iota = lax.broadcasted_iota(jnp.int32, d.shape, 1)
    idx = jnp.min(jnp.where(d == md, iota, jnp.int32(2 ** 30)),
                  axis=1, keepdims=True)
    idx_ref[...] = idx
    idx128_ref[...] = jnp.reshape(idx[:, 0], (1, TB // 128, 128))
    sel = iota == idx
    yt_ref[...] = jnp.sum(jnp.where(sel, scores, 0.0), axis=1, keepdims=True)
    wl_ref[...] = jnp.sum(
        jnp.where(sel, cbt_ref[INPUT_DIM:INPUT_DIM + 1, :], 0.0),
        axis=1, keepdims=True)


def _tc_call(x, W1, b1, W2, b2, W3, b3, cbt):
    grid = (CB // TB,)
    full = lambda shape: pl.BlockSpec(shape, lambda i: (0, 0))
    return pl.pallas_call(
        _tc_body,
        grid=grid,
        in_specs=[
            pl.BlockSpec((TB, INPUT_DIM), lambda i: (i, 0)),
            full((INPUT_DIM, 256)),
            full((1, 256)),
            full((256, 256)),
            full((1, 256)),
            full((256, AUG)),
            full((1, AUG)),
            full((AUG, EXPERT_NUM)),
        ],
        out_specs=[
            pl.BlockSpec((TB, AUG), lambda i: (i, 0)),
            pl.BlockSpec((TB, 1), lambda i: (i, 0)),
            pl.BlockSpec((1, TB // 128, 128), lambda i: (i, 0, 0)),
            pl.BlockSpec((TB, 1), lambda i: (i, 0)),
            pl.BlockSpec((TB, 1), lambda i: (i, 0)),
        ],
        out_shape=[
            jax.ShapeDtypeStruct((CB, AUG), jnp.float32),
            jax.ShapeDtypeStruct((CB, 1), jnp.int32),
            jax.ShapeDtypeStruct((CB // TB, TB // 128, 128), jnp.int32),
            jax.ShapeDtypeStruct((CB, 1), jnp.float32),
            jax.ShapeDtypeStruct((CB, 1), jnp.float32),
        ],
    )(x, W1, b1, W2, b2, W3, b3, cbt)


def _make_sc_gather():
    from jax.experimental.pallas import tpu_sc as plsc

    info = plsc.get_sparse_core_info()
    nc = info.num_cores
    mesh = plsc.VectorSubcoreMesh(core_axis_name="c", subcore_axis_name="s")

    @functools.partial(
        pl.kernel, mesh=mesh,
        out_type=jax.ShapeDtypeStruct((CB, INPUT_DIM), jnp.float32),
        scratch_types=[
            pltpu.VMEM((NCHUNK, 128), jnp.int32),
            pltpu.VMEM((BPW, INPUT_DIM), jnp.float32),
            pltpu.SemaphoreType.DMA,
        ],
        compiler_params=pltpu.CompilerParams(use_tc_tiling_on_sc=True),
    )
    def gather_k(table_hbm, idx_hbm, out_hbm, idx_v, rows_v, sem):
        wid = lax.axis_index("s") * nc + lax.axis_index("c")
        pltpu.sync_copy(idx_hbm.at[wid], idx_v)
        copies = [
            pltpu.async_copy(table_hbm.at[idx_v.at[j]],
                             rows_v.at[pl.ds(j * 128, 128)], sem)
            for j in range(NCHUNK)
        ]
        for c in copies:
            c.wait()
        pltpu.sync_copy(rows_v, out_hbm.at[pl.ds(wid * BPW, BPW)])

    return gather_k


def kernel(input_tensor, W1, b1, W2, b2, W3, b3, codebook):
    cbt = codebook.T
    table = codebook[:, :INPUT_DIM]
    gather = _make_sc_gather()
    b1r, b2r, b3r = b1.reshape(1, 256), b2.reshape(1, 256), b3.reshape(1, AUG)

    w_hat, idx2, idx128, y_tilde, w_last = _tc_call(
        input_tensor, W1, b1r, W2, b2r, W3, b3r, cbt)
    g = gather(table, idx128)
    w_tilde = jnp.concatenate([g, w_last], axis=1)
    return (w_hat, w_tilde, idx2, y_tilde)
